# baseline (device time: 1499741 ns/iter reference)
import jax
import jax.numpy as jnp
from jax import lax
from jax.experimental import pallas as pl
from jax.experimental.pallas import tpu as pltpu

N_DEV = 32


def kernel(x, w_mat):
    m, k = x.shape
    _, n = w_mat.shape
    m_blk = m // N_DEV

    def body(x_ref, w_ref, out_ref, send_buf, recv_buf,
             send_sems, recv_sems, credit_sem):
        my = lax.axis_index("i")
        left = (my - 1) % N_DEV
        right = (my + 1) % N_DEV

        barrier_sem = pltpu.get_barrier_semaphore()
        for nbr in (left, right):
            pl.semaphore_signal(
                barrier_sem, inc=1,
                device_id=(nbr,), device_id_type=pl.DeviceIdType.MESH,
            )
        pl.semaphore_wait(barrier_sem, 2)

        def partial(b):
            xb = x_ref[pl.ds(b * m_blk, m_blk), :]
            return jnp.dot(xb, w_ref[...], preferred_element_type=jnp.float32)

        def hop(slot, src_buf):
            rdma = pltpu.make_async_remote_copy(
                src_ref=src_buf.at[slot],
                dst_ref=recv_buf.at[slot],
                send_sem=send_sems.at[slot],
                recv_sem=recv_sems.at[slot],
                device_id=(right,),
                device_id_type=pl.DeviceIdType.MESH,
            )
            rdma.start()
            rdma.wait()

        send_buf[0] = partial((my - 1) % N_DEV)
        hop(0, send_buf)

        for s in range(1, N_DEV):
            rs = (s - 1) % 2
            acc = recv_buf[rs] + partial((my - 1 - s) % N_DEV)
            if s <= N_DEV - 3:
                pl.semaphore_signal(
                    credit_sem, inc=1,
                    device_id=(left,), device_id_type=pl.DeviceIdType.MESH,
                )
            if s <= N_DEV - 2:
                ss = s % 2
                send_buf[ss] = acc
                if s >= 2:
                    pl.semaphore_wait(credit_sem, 1)
                hop(ss, send_buf)
            else:
                out_ref[...] = acc * jax.nn.sigmoid(acc)

    return pl.pallas_call(
        body,
        out_shape=jax.ShapeDtypeStruct((m_blk, n), jnp.float32),
        in_specs=[
            pl.BlockSpec(memory_space=pltpu.VMEM),
            pl.BlockSpec(memory_space=pltpu.VMEM),
        ],
        out_specs=pl.BlockSpec(memory_space=pltpu.VMEM),
        scratch_shapes=[
            pltpu.VMEM((2, m_blk, n), jnp.float32),
            pltpu.VMEM((2, m_blk, n), jnp.float32),
            pltpu.SemaphoreType.DMA((2,)),
            pltpu.SemaphoreType.DMA((2,)),
            pltpu.SemaphoreType.REGULAR,
        ],
        compiler_params=pltpu.CompilerParams(collective_id=0),
    )(x, w_mat)


# device time: 1478771 ns/iter; 1.0142x vs baseline; 1.0142x over previous
import jax
import jax.numpy as jnp
from jax import lax
from jax.experimental import pallas as pl
from jax.experimental.pallas import tpu as pltpu

N_DEV = 32


def kernel(x, w_mat):
    m, k = x.shape
    _, n = w_mat.shape
    m_blk = m // N_DEV
    n2 = n // 2

    def body(x_ref, w_ref, out_ref,
             send_r, recv_r, send_l, recv_l,
             send_sems_r, recv_sems_r, send_sems_l, recv_sems_l,
             credit_r, credit_l):
        my = lax.axis_index("i")
        left = (my - 1) % N_DEV
        right = (my + 1) % N_DEV

        barrier_sem = pltpu.get_barrier_semaphore()
        for nbr in (left, right):
            pl.semaphore_signal(
                barrier_sem, inc=1,
                device_id=(nbr,), device_id_type=pl.DeviceIdType.MESH,
            )
        pl.semaphore_wait(barrier_sem, 2)

        def partial_r(s):
            b = (my - 1 - s) % N_DEV
            xb = x_ref[pl.ds(b * m_blk, m_blk), :]
            return jnp.dot(xb, w_ref[:, :n2], preferred_element_type=jnp.float32)

        def partial_l(s):
            b = (my + 1 + s) % N_DEV
            xb = x_ref[pl.ds(b * m_blk, m_blk), :]
            return jnp.dot(xb, w_ref[:, n2:], preferred_element_type=jnp.float32)

        def start_hops(slot):
            rdma_r = pltpu.make_async_remote_copy(
                src_ref=send_r.at[slot], dst_ref=recv_r.at[slot],
                send_sem=send_sems_r.at[slot], recv_sem=recv_sems_r.at[slot],
                device_id=(right,), device_id_type=pl.DeviceIdType.MESH,
            )
            rdma_l = pltpu.make_async_remote_copy(
                src_ref=send_l.at[slot], dst_ref=recv_l.at[slot],
                send_sem=send_sems_l.at[slot], recv_sem=recv_sems_l.at[slot],
                device_id=(left,), device_id_type=pl.DeviceIdType.MESH,
            )
            rdma_r.start()
            rdma_l.start()
            return rdma_r, rdma_l

        send_r[0] = partial_r(0)
        send_l[0] = partial_l(0)
        in_flight = start_hops(0)

        for s in range(1, N_DEV):
            rs = (s - 1) % 2
            pr = partial_r(s)
            pl_ = partial_l(s)
            rdma_r, rdma_l = in_flight
            rdma_r.wait()
            rdma_l.wait()
            acc_r = recv_r[rs] + pr
            acc_l = recv_l[rs] + pl_
            if s <= N_DEV - 3:
                pl.semaphore_signal(
                    credit_r, inc=1,
                    device_id=(left,), device_id_type=pl.DeviceIdType.MESH,
                )
                pl.semaphore_signal(
                    credit_l, inc=1,
                    device_id=(right,), device_id_type=pl.DeviceIdType.MESH,
                )
            if s <= N_DEV - 2:
                ss = s % 2
                send_r[ss] = acc_r
                send_l[ss] = acc_l
                if s >= 2:
                    pl.semaphore_wait(credit_r, 1)
                    pl.semaphore_wait(credit_l, 1)
                in_flight = start_hops(ss)
            else:
                out_ref[:, :n2] = acc_r * jax.nn.sigmoid(acc_r)
                out_ref[:, n2:] = acc_l * jax.nn.sigmoid(acc_l)

    return pl.pallas_call(
        body,
        out_shape=jax.ShapeDtypeStruct((m_blk, n), jnp.float32),
        in_specs=[
            pl.BlockSpec(memory_space=pltpu.VMEM),
            pl.BlockSpec(memory_space=pltpu.VMEM),
        ],
        out_specs=pl.BlockSpec(memory_space=pltpu.VMEM),
        scratch_shapes=[
            pltpu.VMEM((2, m_blk, n2), jnp.float32),
            pltpu.VMEM((2, m_blk, n2), jnp.float32),
            pltpu.VMEM((2, m_blk, n2), jnp.float32),
            pltpu.VMEM((2, m_blk, n2), jnp.float32),
            pltpu.SemaphoreType.DMA((2,)),
            pltpu.SemaphoreType.DMA((2,)),
            pltpu.SemaphoreType.DMA((2,)),
            pltpu.SemaphoreType.DMA((2,)),
            pltpu.SemaphoreType.REGULAR,
            pltpu.SemaphoreType.REGULAR,
        ],
        compiler_params=pltpu.CompilerParams(collective_id=0),
    )(x, w_mat)


# device time: 789346 ns/iter; 1.9000x vs baseline; 1.8734x over previous
import jax
import jax.numpy as jnp
from jax import lax
from jax.experimental import pallas as pl
from jax.experimental.pallas import tpu as pltpu

N_DEV = 32

_PLANE_ORDER = [(0, 0), (1, 0), (1, 1), (0, 1), (0, 2), (1, 2), (1, 3), (0, 3)]
_POS_TO_COORD = [
    (x, y, p // 8) for p in range(N_DEV) for (x, y) in [_PLANE_ORDER[p % 8]]
]
_COORD_TO_POS = {c: p for p, c in enumerate(_POS_TO_COORD)}

_CELLS = [
    (0, 0), (1, 0), (2, 0), (3, 0), (3, 1), (2, 1), (1, 1), (1, 2),
    (2, 2), (3, 2), (3, 3), (2, 3), (1, 3), (0, 3), (0, 2), (0, 1),
]
_HAM = []
for _i, (_y, _z) in enumerate(_CELLS):
    _xs = (0, 1) if _i % 2 == 0 else (1, 0)
    _HAM += [(_x, _y, _z) for _x in _xs]
assert len(set(_HAM)) == N_DEV
for _j in range(N_DEV):
    _a, _b = _HAM[_j], _HAM[(_j + 1) % N_DEV]
    assert sum(abs(_p - _q) for _p, _q in zip(_a, _b)) == 1, (_a, _b)

_RING = [_COORD_TO_POS[c] for c in _HAM]
_RIDX = {p: j for j, p in enumerate(_RING)}

_SUCC = [_RING[(_RIDX[p] + 1) % N_DEV] for p in range(N_DEV)]
_PRED = [_RING[(_RIDX[p] - 1) % N_DEV] for p in range(N_DEV)]
_B_F = [[_RING[(_RIDX[p] - 1 - s) % N_DEV] for s in range(N_DEV)]
        for p in range(N_DEV)]
_B_R = [[_RING[(_RIDX[p] + 1 + s) % N_DEV] for s in range(N_DEV)]
        for p in range(N_DEV)]


def kernel(x, w_mat):
    m, k = x.shape
    _, n = w_mat.shape
    m_blk = m // N_DEV
    n2 = n // 2

    my = lax.axis_index("i")
    meta = jnp.concatenate([
        jnp.take(jnp.array(_SUCC, jnp.int32), my)[None],
        jnp.take(jnp.array(_PRED, jnp.int32), my)[None],
        jnp.take(jnp.array(_B_F, jnp.int32), my, axis=0),
        jnp.take(jnp.array(_B_R, jnp.int32), my, axis=0),
    ])

    def body(meta_ref, x_ref, w_ref, out_ref,
             send_f, recv_f, send_r, recv_r,
             send_sems_f, recv_sems_f, send_sems_r, recv_sems_r,
             credit_f, credit_r):
        succ = meta_ref[0]
        pred = meta_ref[1]

        barrier_sem = pltpu.get_barrier_semaphore()
        for nbr in (pred, succ):
            pl.semaphore_signal(
                barrier_sem, inc=1,
                device_id=(nbr,), device_id_type=pl.DeviceIdType.MESH,
            )
        pl.semaphore_wait(barrier_sem, 2)

        def partial_f(s):
            b = meta_ref[2 + s]
            xb = x_ref[pl.ds(b * m_blk, m_blk), :]
            return jnp.dot(xb, w_ref[:, :n2], preferred_element_type=jnp.float32)

        def partial_r(s):
            b = meta_ref[2 + N_DEV + s]
            xb = x_ref[pl.ds(b * m_blk, m_blk), :]
            return jnp.dot(xb, w_ref[:, n2:], preferred_element_type=jnp.float32)

        def start_hops(slot):
            rdma_f = pltpu.make_async_remote_copy(
                src_ref=send_f.at[slot], dst_ref=recv_f.at[slot],
                send_sem=send_sems_f.at[slot], recv_sem=recv_sems_f.at[slot],
                device_id=(succ,), device_id_type=pl.DeviceIdType.MESH,
            )
            rdma_r = pltpu.make_async_remote_copy(
                src_ref=send_r.at[slot], dst_ref=recv_r.at[slot],
                send_sem=send_sems_r.at[slot], recv_sem=recv_sems_r.at[slot],
                device_id=(pred,), device_id_type=pl.DeviceIdType.MESH,
            )
            rdma_f.start()
            rdma_r.start()
            return rdma_f, rdma_r

        send_f[0] = partial_f(0)
        send_r[0] = partial_r(0)
        in_flight = start_hops(0)

        for s in range(1, N_DEV):
            rs = (s - 1) % 2
            pf = partial_f(s)
            pr = partial_r(s)
            rdma_f, rdma_r = in_flight
            rdma_f.wait()
            rdma_r.wait()
            acc_f = recv_f[rs] + pf
            acc_r = recv_r[rs] + pr
            if s <= N_DEV - 3:
                pl.semaphore_signal(
                    credit_f, inc=1,
                    device_id=(pred,), device_id_type=pl.DeviceIdType.MESH,
                )
                pl.semaphore_signal(
                    credit_r, inc=1,
                    device_id=(succ,), device_id_type=pl.DeviceIdType.MESH,
                )
            if s <= N_DEV - 2:
                ss = s % 2
                send_f[ss] = acc_f
                send_r[ss] = acc_r
                if s >= 2:
                    pl.semaphore_wait(credit_f, 1)
                    pl.semaphore_wait(credit_r, 1)
                in_flight = start_hops(ss)
            else:
                out_ref[:, :n2] = acc_f * jax.nn.sigmoid(acc_f)
                out_ref[:, n2:] = acc_r * jax.nn.sigmoid(acc_r)

    return pl.pallas_call(
        body,
        out_shape=jax.ShapeDtypeStruct((m_blk, n), jnp.float32),
        in_specs=[
            pl.BlockSpec(memory_space=pltpu.SMEM),
            pl.BlockSpec(memory_space=pltpu.VMEM),
            pl.BlockSpec(memory_space=pltpu.VMEM),
        ],
        out_specs=pl.BlockSpec(memory_space=pltpu.VMEM),
        scratch_shapes=[
            pltpu.VMEM((2, m_blk, n2), jnp.float32),
            pltpu.VMEM((2, m_blk, n2), jnp.float32),
            pltpu.VMEM((2, m_blk, n2), jnp.float32),
            pltpu.VMEM((2, m_blk, n2), jnp.float32),
            pltpu.SemaphoreType.DMA((2,)),
            pltpu.SemaphoreType.DMA((2,)),
            pltpu.SemaphoreType.DMA((2,)),
            pltpu.SemaphoreType.DMA((2,)),
            pltpu.SemaphoreType.REGULAR,
            pltpu.SemaphoreType.REGULAR,
        ],
        compiler_params=pltpu.CompilerParams(collective_id=0),
    )(meta, x, w_mat)


# device time: 715102 ns/iter; 2.0972x vs baseline; 1.1038x over previous
import jax
import jax.numpy as jnp
from jax import lax
from jax.experimental import pallas as pl
from jax.experimental.pallas import tpu as pltpu

N_DEV = 32

_PLANE_ORDER = [(0, 0), (1, 0), (1, 1), (0, 1), (0, 2), (1, 2), (1, 3), (0, 3)]
_POS_TO_COORD = [
    (x, y, p // 8) for p in range(N_DEV) for (x, y) in [_PLANE_ORDER[p % 8]]
]
_COORD_TO_POS = {c: p for p, c in enumerate(_POS_TO_COORD)}

_CELLS = [
    (0, 0), (1, 0), (2, 0), (3, 0), (3, 1), (2, 1), (1, 1), (1, 2),
    (2, 2), (3, 2), (3, 3), (2, 3), (1, 3), (0, 3), (0, 2), (0, 1),
]
_HAM = []
for _i, (_y, _z) in enumerate(_CELLS):
    _xs = (0, 1) if _i % 2 == 0 else (1, 0)
    _HAM += [(_x, _y, _z) for _x in _xs]
assert len(set(_HAM)) == N_DEV
for _j in range(N_DEV):
    _a, _b = _HAM[_j], _HAM[(_j + 1) % N_DEV]
    assert sum(abs(_p - _q) for _p, _q in zip(_a, _b)) == 1, (_a, _b)

_RING = [_COORD_TO_POS[c] for c in _HAM]
_RIDX = {p: j for j, p in enumerate(_RING)}

_SUCC = [_RING[(_RIDX[p] + 1) % N_DEV] for p in range(N_DEV)]
_PRED = [_RING[(_RIDX[p] - 1) % N_DEV] for p in range(N_DEV)]
_B_F = [[_RING[(_RIDX[p] - 1 - s) % N_DEV] for s in range(N_DEV)]
        for p in range(N_DEV)]
_B_R = [[_RING[(_RIDX[p] + 1 + s) % N_DEV] for s in range(N_DEV)]
        for p in range(N_DEV)]


def kernel(x, w_mat):
    m, k = x.shape
    _, n = w_mat.shape
    m_blk = m // N_DEV
    n2 = n // 2
    q = n2 // 2

    my = lax.axis_index("i")
    meta = jnp.concatenate([
        jnp.take(jnp.array(_SUCC, jnp.int32), my)[None],
        jnp.take(jnp.array(_PRED, jnp.int32), my)[None],
        jnp.take(jnp.array(_B_F, jnp.int32), my, axis=0),
        jnp.take(jnp.array(_B_R, jnp.int32), my, axis=0),
    ])

    def body(meta_ref, x_ref, w_ref, out_ref,
             send_f, recv_f, send_r, recv_r,
             send_sems_f, recv_sems_f, send_sems_r, recv_sems_r,
             credit_f, credit_r):
        succ = meta_ref[0]
        pred = meta_ref[1]

        barrier_sem = pltpu.get_barrier_semaphore()
        for nbr in (pred, succ):
            pl.semaphore_signal(
                barrier_sem, inc=1,
                device_id=(nbr,), device_id_type=pl.DeviceIdType.MESH,
            )
        pl.semaphore_wait(barrier_sem, 2)

        def partial_f(s):
            b = meta_ref[2 + s]
            xb = x_ref[pl.ds(b * m_blk, m_blk), :]
            return jnp.dot(xb, w_ref[:, :n2], preferred_element_type=jnp.float32)

        def partial_r(s):
            b = meta_ref[2 + N_DEV + s]
            xb = x_ref[pl.ds(b * m_blk, m_blk), :]
            return jnp.dot(xb, w_ref[:, n2:], preferred_element_type=jnp.float32)

        def mk_rdma(slot, c, fwd):
            sl = slice(c * q, (c + 1) * q)
            if fwd:
                return pltpu.make_async_remote_copy(
                    src_ref=send_f.at[slot, :, sl], dst_ref=recv_f.at[slot, :, sl],
                    send_sem=send_sems_f.at[slot, c],
                    recv_sem=recv_sems_f.at[slot, c],
                    device_id=(succ,), device_id_type=pl.DeviceIdType.MESH,
                )
            return pltpu.make_async_remote_copy(
                src_ref=send_r.at[slot, :, sl], dst_ref=recv_r.at[slot, :, sl],
                send_sem=send_sems_r.at[slot, c],
                recv_sem=recv_sems_r.at[slot, c],
                device_id=(pred,), device_id_type=pl.DeviceIdType.MESH,
            )

        pf = partial_f(0)
        pr = partial_r(0)
        flight = {}
        for c in (0, 1):
            sl = slice(c * q, (c + 1) * q)
            send_f[0, :, sl] = pf[:, sl]
            flight["f", c] = mk_rdma(0, c, True)
            flight["f", c].start()
            send_r[0, :, sl] = pr[:, sl]
            flight["r", c] = mk_rdma(0, c, False)
            flight["r", c].start()

        for s in range(1, N_DEV):
            rs = (s - 1) % 2
            ss = s % 2
            pf = partial_f(s)
            pr = partial_r(s)
            acc = {}
            for c in (0, 1):
                sl = slice(c * q, (c + 1) * q)
                for d, buf_recv, buf_send, credit in (
                    ("f", recv_f, send_f, credit_f),
                    ("r", recv_r, send_r, credit_r),
                ):
                    flight[d, c].wait()
                    a = buf_recv[rs, :, sl] + (pf if d == "f" else pr)[:, sl]
                    if s <= N_DEV - 2:
                        buf_send[ss, :, sl] = a
                        if c == 0 and s >= 2:
                            pl.semaphore_wait(credit, 1)
                        flight[d, c] = mk_rdma(ss, c, d == "f")
                        flight[d, c].start()
                    else:
                        acc[d, c] = a
            if s <= N_DEV - 3:
                pl.semaphore_signal(
                    credit_f, inc=1,
                    device_id=(pred,), device_id_type=pl.DeviceIdType.MESH,
                )
                pl.semaphore_signal(
                    credit_r, inc=1,
                    device_id=(succ,), device_id_type=pl.DeviceIdType.MESH,
                )
            if s == N_DEV - 1:
                for c in (0, 1):
                    af = acc["f", c]
                    ar = acc["r", c]
                    out_ref[:, c * q:(c + 1) * q] = af * jax.nn.sigmoid(af)
                    out_ref[:, n2 + c * q:n2 + (c + 1) * q] = (
                        ar * jax.nn.sigmoid(ar)
                    )

    return pl.pallas_call(
        body,
        out_shape=jax.ShapeDtypeStruct((m_blk, n), jnp.float32),
        in_specs=[
            pl.BlockSpec(memory_space=pltpu.SMEM),
            pl.BlockSpec(memory_space=pltpu.VMEM),
            pl.BlockSpec(memory_space=pltpu.VMEM),
        ],
        out_specs=pl.BlockSpec(memory_space=pltpu.VMEM),
        scratch_shapes=[
            pltpu.VMEM((2, m_blk, n2), jnp.float32),
            pltpu.VMEM((2, m_blk, n2), jnp.float32),
            pltpu.VMEM((2, m_blk, n2), jnp.float32),
            pltpu.VMEM((2, m_blk, n2), jnp.float32),
            pltpu.SemaphoreType.DMA((2, 2)),
            pltpu.SemaphoreType.DMA((2, 2)),
            pltpu.SemaphoreType.DMA((2, 2)),
            pltpu.SemaphoreType.DMA((2, 2)),
            pltpu.SemaphoreType.REGULAR,
            pltpu.SemaphoreType.REGULAR,
        ],
        compiler_params=pltpu.CompilerParams(collective_id=0),
    )(meta, x, w_mat)
